# Initial kernel scaffold; baseline (speedup 1.0000x reference)
import jax
import jax.numpy as jnp
from jax.experimental import pallas as pl


def kernel(llr_in, cn_weight, ch_weight, edge_to_vn, edge_to_cn):
    raise NotImplementedError("write your pallas kernel here")


# speedup vs baseline: 12.3813x; 12.3813x over previous
import functools

import jax
import jax.numpy as jnp
from jax import lax
from jax.experimental import pallas as pl
from jax.experimental.pallas import tpu as pltpu
from jax.experimental.pallas import tpu_sc as plsc

N = 26112
M = 17664
E = 121344
CLIP = 20.0
B = 64
BH = 32
NS = 16
K7 = E % M
CHUNK = 48
CH7 = K7 // CHUNK
CH_TOT = M // CHUNK
CPT = CH_TOT // NS
NPT = N // NS
SLAB = NPT // 4
BIG = 1e9


def _minset(vals):
    m = vals[0]
    for v in vals[1:]:
        m = jnp.minimum(m, v)
    return m


def _body(iter0, src_hbm, llr_hbm, wpack_hbm, vn_hbm, c2v_in_hbm,
          marg_out, c2v_out, part_out,
          wv, lA, lB, i7, g7, i6, g6, gbuf, cbuf, accb, Bacc):
    c = lax.axis_index("c")
    s = lax.axis_index("s")
    pltpu.sync_copy(wpack_hbm, wv)
    winit = wv[pl.ds(0, 16)]
    wcn = wv[pl.ds(16, 16)]
    wg = wv[pl.ds(32, 16)]

    zero = jnp.zeros((16,), jnp.float32)

    def _zrow(i, _):
        lA[i, pl.ds(0, 16)] = zero
        lA[i, pl.ds(16, 16)] = zero
        return 0

    lax.fori_loop(0, SLAB, _zrow, 0)
    rbase = s * NPT
    for q in range(4):
        pltpu.sync_copy(lA, Bacc.at[pl.ds(rbase + q * SLAB, SLAB)])
    plsc.subcore_barrier()

    coff = c * N

    def process_chunk(cj, deg, ir, gi, R):
        Ec = CHUNK * deg
        if deg == 7:
            ebase = 336 * cj
        else:
            ebase = 107520 + 288 * (cj - 320)
        for r in range(3):
            pltpu.sync_copy(vn_hbm.at[pl.ds(ebase + r * R, R)], ir.at[r])
        for r in range(3):
            for kk in range(R // 16):
                gi[r, pl.ds(kk * 16, 16)] = ir[r, pl.ds(kk * 16, 16)] + coff
        for r in range(3):
            pltpu.sync_copy(src_hbm.at[gi.at[r]], gbuf.at[pl.ds(r * R, R)])
        if not iter0:
            pltpu.sync_copy(c2v_in_hbm.at[pl.ds(c * E + ebase, Ec)],
                            cbuf.at[pl.ds(0, Ec)])

        def cn_body(i, _):
            rb = i * deg
            for h in range(2):
                dsh = pl.ds(16 * h, 16)
                avs, sgs, gts = [], [], []
                for d in range(deg):
                    g = gbuf[rb + d, dsh]
                    if iter0:
                        v = g * wg
                    else:
                        v = g - cbuf[rb + d, dsh]
                    avs.append(jnp.minimum(jnp.abs(v), CLIP))
                    sgs.append(jnp.where(v >= 0.0, 1.0, -1.0))
                m1 = _minset(avs)
                ts = []
                for d in range(deg):
                    gt = avs[d] > m1
                    gts.append(gt)
                    ts.append(jnp.where(gt, avs[d], BIG))
                m2 = _minset(ts)
                p = sgs[0]
                for d in range(1, deg):
                    p = p * sgs[d]
                pw = p * wcn
                for d in range(deg):
                    e = jnp.where(gts[d], m1, m2)
                    x = (pw * e) * sgs[d]
                    x = jnp.minimum(jnp.maximum(x, -CLIP), CLIP)
                    cbuf[rb + d, dsh] = x
            return 0

        lax.fori_loop(0, CHUNK, cn_body, 0)
        pltpu.sync_copy(cbuf.at[pl.ds(0, Ec)],
                        c2v_out.at[pl.ds(c * E + ebase, Ec)])
        for r in range(3):
            pltpu.sync_copy(cbuf.at[pl.ds(r * R, R)], Bacc.at[ir.at[r]],
                            add=True)

    t0 = s * CPT
    n7 = jnp.clip(CH7 - t0, 0, CPT)

    def do7(j, _):
        process_chunk(t0 + j, 7, i7, g7, 112)
        return 0

    def do6(j, _):
        process_chunk(t0 + j, 6, i6, g6, 96)
        return 0

    lax.fori_loop(0, n7, do7, 0)
    lax.fori_loop(n7, CPT, do6, 0)
    plsc.subcore_barrier()

    acc = jnp.zeros((16,), jnp.float32)
    for q in range(4):
        r0 = rbase + q * SLAB
        pltpu.sync_copy(Bacc.at[pl.ds(r0, SLAB)], lA)
        pltpu.sync_copy(llr_hbm.at[pl.ds(c * N + r0, SLAB), :], lB)

        def wrow(i, a2):
            for h in range(2):
                dsh = pl.ds(16 * h, 16)
                m = lA[i, dsh] + lB[i, dsh] * winit
                lA[i, dsh] = m
                a2 = a2 + 1.0 / (1.0 + jnp.exp(m))
            return a2

        acc = lax.fori_loop(0, SLAB, wrow, acc)
        pltpu.sync_copy(lA, marg_out.at[pl.ds(c * N + r0, SLAB), :])
    accb[...] = acc
    plsc.subcore_barrier()
    pltpu.sync_copy(accb, part_out.at[s * 2 + c])


def _make_call(iter0):
    mesh = plsc.VectorSubcoreMesh(core_axis_name="c", subcore_axis_name="s")
    out_type = (
        jax.ShapeDtypeStruct((2 * N, BH), jnp.float32),
        jax.ShapeDtypeStruct((2 * E, BH), jnp.float32),
        jax.ShapeDtypeStruct((NS * 2, 16), jnp.float32),
    )
    scratch = [
        pltpu.VMEM((48,), jnp.float32),
        pltpu.VMEM((SLAB, BH), jnp.float32),
        pltpu.VMEM((SLAB, BH), jnp.float32),
        pltpu.VMEM((3, 112), jnp.int32),
        pltpu.VMEM((3, 112), jnp.int32),
        pltpu.VMEM((3, 96), jnp.int32),
        pltpu.VMEM((3, 96), jnp.int32),
        pltpu.VMEM((336, BH), jnp.float32),
        pltpu.VMEM((336, BH), jnp.float32),
        pltpu.VMEM((16,), jnp.float32),
        pltpu.VMEM_SHARED((N, BH), jnp.float32),
    ]
    body = functools.partial(_body, iter0)
    return pl.kernel(body, out_type=out_type, mesh=mesh,
                     scratch_types=scratch,
                     compiler_params=pltpu.CompilerParams(
                         use_tc_tiling_on_sc=False),
                     name="ldpc_sc_it0" if iter0 else "ldpc_sc")


def kernel(llr_in, cn_weight, ch_weight, edge_to_vn, edge_to_cn):
    iters = int(cn_weight.shape[0])
    llr2 = llr_in.T.reshape(N, 2, BH).transpose(1, 0, 2).reshape(2 * N, BH)
    vn = edge_to_vn.astype(jnp.int32)
    one = jnp.ones((16,), jnp.float32)

    def wpack(it):
        winit = ch_weight[it + 1] if it + 1 < iters else jnp.float32(1.0)
        return jnp.concatenate([one * winit, one * cn_weight[it],
                                one * ch_weight[it]])

    call0 = _make_call(True)
    call = _make_call(False)
    dummy = jnp.zeros((8, BH), jnp.float32)

    marg, c2v, parts = call0(llr2, llr2 + 0.0, wpack(0), vn, dummy)
    for it in range(1, iters):
        marg, c2v, parts = call(marg, llr2, wpack(it), vn, c2v)
    return jnp.sum(parts) / (B * N)


# speedup vs baseline: 13.0407x; 1.0533x over previous
import functools

import jax
import jax.numpy as jnp
from jax import lax
from jax.experimental import pallas as pl
from jax.experimental.pallas import tpu as pltpu
from jax.experimental.pallas import tpu_sc as plsc

N = 26112
M = 17664
E = 121344
CLIP = 20.0
B = 64
BH = 32
NS = 16
K7 = E % M
CHUNK = 48
CH7 = K7 // CHUNK
CH_TOT = M // CHUNK
CPT = CH_TOT // NS
NPT = N // NS
SLAB = NPT // 4
E7 = K7 * 7
BIG = 1e9


def _minset(vals):
    m = vals[0]
    for v in vals[1:]:
        m = jnp.minimum(m, v)
    return m


def _body(iter0, last, src_hbm, llr_hbm, wpack_hbm, vn7_hbm, vn6_hbm,
          c2v_in_hbm, marg_out, c2v_out, part_out,
          wv, lA, lB, i7, g7, i6, g6, gbuf, cbuf, accb, Bacc,
          semg, semw, semi, semc):
    c = lax.axis_index("c")
    s = lax.axis_index("s")
    pltpu.sync_copy(wpack_hbm, wv)
    winit = wv[pl.ds(0, 16)]
    wcn = wv[pl.ds(16, 16)]
    wg = wv[pl.ds(32, 16)]

    zero = jnp.zeros((16,), jnp.float32)

    def _zrow(i, _):
        lA[i, pl.ds(0, 16)] = zero
        lA[i, pl.ds(16, 16)] = zero
        return 0

    lax.fori_loop(0, SLAB, _zrow, 0)
    rbase = s * NPT
    for q in range(4):
        pltpu.sync_copy(lA, Bacc.at[pl.ds(rbase + q * SLAB, SLAB)])
    plsc.subcore_barrier()

    def process_chunk(cj, deg, ir, gi, vh, R, nrow):
        Ec = CHUNK * deg
        if deg == 7:
            ebase = 336 * cj
            rowbase = 3 * cj
        else:
            ebase = E7 + 288 * (cj - CH7)
            rowbase = 3 * (cj - CH7)
        pltpu.sync_copy(vh.at[pl.ds(rowbase, 3)], ir)
        pltpu.sync_copy(vh.at[pl.ds(c * nrow + rowbase, 3)], gi)
        if not iter0:
            pltpu.sync_copy(c2v_in_hbm.at[pl.ds(c * E + ebase, Ec)],
                            cbuf.at[pl.ds(0, Ec)])
        for r in range(3):
            pltpu.sync_copy(src_hbm.at[gi.at[r]], gbuf.at[pl.ds(r * R, R)])

        def cn_body(i, _):
            rb = i * deg
            for h in range(2):
                dsh = pl.ds(16 * h, 16)
                avs, sgs, gts = [], [], []
                for d in range(deg):
                    g = gbuf[rb + d, dsh]
                    if iter0:
                        v = g * wg
                    else:
                        v = g - cbuf[rb + d, dsh]
                    avs.append(jnp.minimum(jnp.abs(v), CLIP))
                    sgs.append(jnp.where(v >= 0.0, 1.0, -1.0))
                m1 = _minset(avs)
                ts = []
                for d in range(deg):
                    gt = avs[d] > m1
                    gts.append(gt)
                    ts.append(jnp.where(gt, avs[d], BIG))
                m2 = _minset(ts)
                p = sgs[0]
                for d in range(1, deg):
                    p = p * sgs[d]
                pw = p * wcn
                for d in range(deg):
                    e = jnp.where(gts[d], m1, m2)
                    x = (pw * e) * sgs[d]
                    x = jnp.minimum(jnp.maximum(x, -CLIP), CLIP)
                    cbuf[rb + d, dsh] = x
            return 0

        lax.fori_loop(0, CHUNK, cn_body, 0)
        pltpu.sync_copy(cbuf.at[pl.ds(0, Ec)],
                        c2v_out.at[pl.ds(c * E + ebase, Ec)])
        for r in range(3):
            pltpu.sync_copy(cbuf.at[pl.ds(r * R, R)], Bacc.at[ir.at[r]],
                            add=True)

    t0 = s * CPT
    n7 = jnp.clip(CH7 - t0, 0, CPT)

    def do7(j, _):
        process_chunk(t0 + j, 7, i7, g7, vn7_hbm, 112, 3 * CH7)
        return 0

    def do6(j, _):
        process_chunk(t0 + j, 6, i6, g6, vn6_hbm, 96, 3 * (CH_TOT - CH7))
        return 0

    lax.fori_loop(0, n7, do7, 0)
    lax.fori_loop(n7, CPT, do6, 0)
    plsc.subcore_barrier()

    acc = jnp.zeros((16,), jnp.float32)
    for q in range(4):
        r0 = rbase + q * SLAB
        pltpu.sync_copy(Bacc.at[pl.ds(r0, SLAB)], lA)
        pltpu.sync_copy(llr_hbm.at[pl.ds(c * N + r0, SLAB), :], lB)

        def wrow(i, a2):
            for h in range(2):
                dsh = pl.ds(16 * h, 16)
                m = lA[i, dsh] + lB[i, dsh] * winit
                if not last:
                    lA[i, dsh] = m
                else:
                    a2 = a2 + 1.0 / (1.0 + jnp.exp(m))
            return a2

        acc = lax.fori_loop(0, SLAB, wrow, acc)
        if not last:
            pltpu.sync_copy(lA, marg_out.at[pl.ds(c * N + r0, SLAB), :])
    accb[...] = acc
    pltpu.sync_copy(accb, part_out.at[s * 2 + c])


def _make_call(iter0, last):
    mesh = plsc.VectorSubcoreMesh(core_axis_name="c", subcore_axis_name="s")
    out_type = (
        jax.ShapeDtypeStruct((2 * N, BH), jnp.float32),
        jax.ShapeDtypeStruct((2 * E, BH), jnp.float32),
        jax.ShapeDtypeStruct((NS * 2, 16), jnp.float32),
    )
    scratch = [
        pltpu.VMEM((48,), jnp.float32),
        pltpu.VMEM((SLAB, BH), jnp.float32),
        pltpu.VMEM((SLAB, BH), jnp.float32),
        pltpu.VMEM((3, 112), jnp.int32),
        pltpu.VMEM((3, 112), jnp.int32),
        pltpu.VMEM((3, 96), jnp.int32),
        pltpu.VMEM((3, 96), jnp.int32),
        pltpu.VMEM((336, BH), jnp.float32),
        pltpu.VMEM((336, BH), jnp.float32),
        pltpu.VMEM((16,), jnp.float32),
        pltpu.VMEM_SHARED((N, BH), jnp.float32),
        pltpu.SemaphoreType.DMA,
        pltpu.SemaphoreType.DMA,
        pltpu.SemaphoreType.DMA,
        pltpu.SemaphoreType.DMA,
    ]
    body = functools.partial(_body, iter0, last)
    return pl.kernel(body, out_type=out_type, mesh=mesh,
                     scratch_types=scratch,
                     compiler_params=pltpu.CompilerParams(
                         use_tc_tiling_on_sc=False),
                     name=f"ldpc_sc_{int(iter0)}{int(last)}")


def kernel(llr_in, cn_weight, ch_weight, edge_to_vn, edge_to_cn):
    iters = int(cn_weight.shape[0])
    llr2 = llr_in.T.reshape(N, 2, BH).transpose(1, 0, 2).reshape(2 * N, BH)
    vn = edge_to_vn.astype(jnp.int32)
    v7 = vn[:E7].reshape(3 * CH7, 112)
    v6 = vn[E7:].reshape(3 * (CH_TOT - CH7), 96)
    vn7 = jnp.concatenate([v7, v7 + N])
    vn6 = jnp.concatenate([v6, v6 + N])
    one = jnp.ones((16,), jnp.float32)

    def wpack(it):
        winit = ch_weight[it + 1] if it + 1 < iters else jnp.float32(1.0)
        return jnp.concatenate([one * winit, one * cn_weight[it],
                                one * ch_weight[it]])

    call0 = _make_call(True, iters == 1)
    call = _make_call(False, False)
    calln = _make_call(False, True)
    dummy = jnp.zeros((8, BH), jnp.float32)

    marg, c2v, parts = call0(llr2, llr2 + 0.0, wpack(0), vn7, vn6, dummy)
    for it in range(1, iters):
        fn = calln if it == iters - 1 else call
        marg, c2v, parts = fn(marg, llr2, wpack(it), vn7, vn6, c2v)
    return jnp.sum(parts) / (B * N)


# speedup vs baseline: 16.8146x; 1.2894x over previous
import functools

import jax
import jax.numpy as jnp
from jax import lax
from jax.experimental import pallas as pl
from jax.experimental.pallas import tpu as pltpu
from jax.experimental.pallas import tpu_sc as plsc

N = 26112
M = 17664
E = 121344
CLIP = 20.0
B = 64
BH = 32
NS = 16
K7 = E % M
CHUNK = 48
CH7 = K7 // CHUNK
CH_TOT = M // CHUNK
CPT = CH_TOT // NS
NPT = N // NS
SLAB = NPT // 4
E7 = K7 * 7
BIG = 1e9


def _minset(vals):
    m = vals[0]
    for v in vals[1:]:
        m = jnp.minimum(m, v)
    return m


def _body(iter0, last, src_hbm, llr_hbm, wpack_hbm, vn7_hbm, vn6_hbm,
          c2v_in_hbm, marg_out, c2v_out, part_out,
          wv, lA, lB, i7, g7, i6, g6, gbuf, cbuf, accb, Bacc,
          semg, semw, semi, semc):
    c = lax.axis_index("c")
    s = lax.axis_index("s")
    pltpu.sync_copy(wpack_hbm, wv)
    winit = wv[pl.ds(0, 16)]
    wcn = wv[pl.ds(16, 16)]
    wg = wv[pl.ds(32, 16)]

    zero = jnp.zeros((16,), jnp.float32)

    def _zrow(i, _):
        lA[i, pl.ds(0, 16)] = zero
        lA[i, pl.ds(16, 16)] = zero
        return 0

    lax.fori_loop(0, SLAB, _zrow, 0)
    rbase = s * NPT
    for q in range(4):
        pltpu.sync_copy(lA, Bacc.at[pl.ds(rbase + q * SLAB, SLAB)])
    plsc.subcore_barrier()

    def process_chunk(cj, deg, ir, gi, vh, R, nrow):
        Ec = CHUNK * deg
        if deg == 7:
            ebase = 336 * cj
            rowbase = 3 * cj
        else:
            ebase = E7 + 288 * (cj - CH7)
            rowbase = 3 * (cj - CH7)
        pltpu.sync_copy(vh.at[pl.ds(rowbase, 3)], ir)
        pltpu.sync_copy(vh.at[pl.ds(c * nrow + rowbase, 3)], gi)
        if not iter0:
            dc = pltpu.async_copy(c2v_in_hbm.at[pl.ds(c * E + ebase, Ec)],
                                  cbuf.at[pl.ds(0, Ec)], semc)
        gs = [pltpu.async_copy(src_hbm.at[gi.at[r]],
                               gbuf.at[pl.ds(r * R, R)], semg)
              for r in range(3)]
        for d in gs:
            d.wait()
        if not iter0:
            dc.wait()

        def cn_body(i, _):
            rb = i * deg
            for h in range(2):
                dsh = pl.ds(16 * h, 16)
                avs, sgs, gts = [], [], []
                for d in range(deg):
                    g = gbuf[rb + d, dsh]
                    if iter0:
                        v = g * wg
                    else:
                        v = g - cbuf[rb + d, dsh]
                    avs.append(jnp.minimum(jnp.abs(v), CLIP))
                    sgs.append(jnp.where(v >= 0.0, 1.0, -1.0))
                m1 = _minset(avs)
                ts = []
                for d in range(deg):
                    gt = avs[d] > m1
                    gts.append(gt)
                    ts.append(jnp.where(gt, avs[d], BIG))
                m2 = _minset(ts)
                p = sgs[0]
                for d in range(1, deg):
                    p = p * sgs[d]
                pw = p * wcn
                for d in range(deg):
                    e = jnp.where(gts[d], m1, m2)
                    x = (pw * e) * sgs[d]
                    x = jnp.minimum(jnp.maximum(x, -CLIP), CLIP)
                    cbuf[rb + d, dsh] = x
            return 0

        lax.fori_loop(0, CHUNK, cn_body, 0)
        wd = pltpu.async_copy(cbuf.at[pl.ds(0, Ec)],
                              c2v_out.at[pl.ds(c * E + ebase, Ec)], semw)
        for r in range(3):
            pltpu.sync_copy(cbuf.at[pl.ds(r * R, R)], Bacc.at[ir.at[r]],
                            add=True)
        wd.wait()

    t0 = s * CPT
    n7 = jnp.clip(CH7 - t0, 0, CPT)

    def do7(j, _):
        process_chunk(t0 + j, 7, i7, g7, vn7_hbm, 112, 3 * CH7)
        return 0

    def do6(j, _):
        process_chunk(t0 + j, 6, i6, g6, vn6_hbm, 96, 3 * (CH_TOT - CH7))
        return 0

    lax.fori_loop(0, n7, do7, 0)
    lax.fori_loop(n7, CPT, do6, 0)
    plsc.subcore_barrier()

    acc = jnp.zeros((16,), jnp.float32)
    for q in range(4):
        r0 = rbase + q * SLAB
        pltpu.sync_copy(Bacc.at[pl.ds(r0, SLAB)], lA)
        pltpu.sync_copy(llr_hbm.at[pl.ds(c * N + r0, SLAB), :], lB)

        def wrow(i, a2):
            for h in range(2):
                dsh = pl.ds(16 * h, 16)
                m = lA[i, dsh] + lB[i, dsh] * winit
                if not last:
                    lA[i, dsh] = m
                else:
                    a2 = a2 + 1.0 / (1.0 + jnp.exp(m))
            return a2

        acc = lax.fori_loop(0, SLAB, wrow, acc)
        if not last:
            pltpu.sync_copy(lA, marg_out.at[pl.ds(c * N + r0, SLAB), :])
    accb[...] = acc
    pltpu.sync_copy(accb, part_out.at[s * 2 + c])


def _make_call(iter0, last):
    mesh = plsc.VectorSubcoreMesh(core_axis_name="c", subcore_axis_name="s")
    out_type = (
        jax.ShapeDtypeStruct((2 * N, BH), jnp.float32),
        jax.ShapeDtypeStruct((2 * E, BH), jnp.float32),
        jax.ShapeDtypeStruct((NS * 2, 16), jnp.float32),
    )
    scratch = [
        pltpu.VMEM((48,), jnp.float32),
        pltpu.VMEM((SLAB, BH), jnp.float32),
        pltpu.VMEM((SLAB, BH), jnp.float32),
        pltpu.VMEM((3, 112), jnp.int32),
        pltpu.VMEM((3, 112), jnp.int32),
        pltpu.VMEM((3, 96), jnp.int32),
        pltpu.VMEM((3, 96), jnp.int32),
        pltpu.VMEM((336, BH), jnp.float32),
        pltpu.VMEM((336, BH), jnp.float32),
        pltpu.VMEM((16,), jnp.float32),
        pltpu.VMEM_SHARED((N, BH), jnp.float32),
        pltpu.SemaphoreType.DMA,
        pltpu.SemaphoreType.DMA,
        pltpu.SemaphoreType.DMA,
        pltpu.SemaphoreType.DMA,
    ]
    body = functools.partial(_body, iter0, last)
    return pl.kernel(body, out_type=out_type, mesh=mesh,
                     scratch_types=scratch,
                     compiler_params=pltpu.CompilerParams(
                         use_tc_tiling_on_sc=False),
                     name=f"ldpc_sc_{int(iter0)}{int(last)}")


def kernel(llr_in, cn_weight, ch_weight, edge_to_vn, edge_to_cn):
    iters = int(cn_weight.shape[0])
    llr2 = llr_in.T.reshape(N, 2, BH).transpose(1, 0, 2).reshape(2 * N, BH)
    vn = edge_to_vn.astype(jnp.int32)
    v7 = vn[:E7].reshape(3 * CH7, 112)
    v6 = vn[E7:].reshape(3 * (CH_TOT - CH7), 96)
    vn7 = jnp.concatenate([v7, v7 + N])
    vn6 = jnp.concatenate([v6, v6 + N])
    one = jnp.ones((16,), jnp.float32)

    def wpack(it):
        winit = ch_weight[it + 1] if it + 1 < iters else jnp.float32(1.0)
        return jnp.concatenate([one * winit, one * cn_weight[it],
                                one * ch_weight[it]])

    call0 = _make_call(True, iters == 1)
    call = _make_call(False, False)
    calln = _make_call(False, True)
    dummy = jnp.zeros((8, BH), jnp.float32)

    marg, c2v, parts = call0(llr2, llr2 + 0.0, wpack(0), vn7, vn6, dummy)
    for it in range(1, iters):
        fn = calln if it == iters - 1 else call
        marg, c2v, parts = fn(marg, llr2, wpack(it), vn7, vn6, c2v)
    return jnp.sum(parts) / (B * N)
